# Initial kernel scaffold; baseline (speedup 1.0000x reference)
import jax
import jax.numpy as jnp
from jax.experimental import pallas as pl


def kernel(features, targets, proxy):
    raise NotImplementedError("write your pallas kernel here")


# speedup vs baseline: 175.6246x; 175.6246x over previous
import jax
import jax.numpy as jnp
from jax.experimental import pallas as pl
from jax.experimental.pallas import tpu as pltpu

NUM_CLASSES = 10
N_PROXY = 16
K = 5
TEMP = 0.05
EPSILON = 0.5
SINKHORN_ITERS = 5
FEAT_DIM = 128
BATCH = 16384

ROW_TILE = 2048
NUM_TILES = BATCH // ROW_TILE


def _rowsum(x):
    return jnp.sum(x, axis=1, keepdims=True)


def _row_lse(x):
    m = jnp.max(x, axis=1, keepdims=True)
    return m + jnp.log(_rowsum(jnp.exp(x - m)))


def _sub_sum(x):
    return jnp.sum(x, axis=0, keepdims=True)


def _sub_lse(x):
    m = jnp.max(x, axis=0, keepdims=True)
    return m + jnp.log(_sub_sum(jnp.exp(x - m)))


def _mpl_kernel(f_ref, oh_ref, proxy_ref, out_ref, C_sc, OH_sc):
    i = pl.program_id(0)

    f = f_ref[...]
    nrm = jnp.sqrt(_rowsum(f * f))
    fn = f / jnp.maximum(nrm, 1e-12)
    g_t = jax.lax.dot_general(proxy_ref[...], fn, (((1,), (1,)), ((), ())),
                              preferred_element_type=jnp.float32)
    oh = oh_ref[...]
    c_t = jnp.zeros((N_PROXY, ROW_TILE), dtype=jnp.float32)
    for c in range(NUM_CLASSES):
        c_t = c_t + oh[c:c + 1, :] * g_t[c * N_PROXY:(c + 1) * N_PROXY, :]
    C_sc[:, pl.ds(i * ROW_TILE, ROW_TILE)] = c_t
    OH_sc[:, pl.ds(i * ROW_TILE, ROW_TILE)] = oh

    @pl.when(i == NUM_TILES - 1)
    def _():
        C = C_sc[...]
        OH = OH_sc[...]

        def sink_iter(_, carry):
            u_prev, v_t = carry
            vrow = jax.lax.dot_general(v_t, OH, (((1,), (0,)), ((), ())),
                                       preferred_element_type=jnp.float32)
            r = _sub_sum(C * vrow)
            u = 1.0 / jnp.maximum(r, 1e-10)
            cu = C * u
            vacc = jax.lax.dot_general(cu, OH, (((1,), (1,)), ((), ())),
                                       preferred_element_type=jnp.float32)
            v_new = 1.0 / jnp.maximum(vacc, 1e-10)
            return u, v_new

        u0 = jnp.ones((1, BATCH), dtype=jnp.float32)
        v0 = jnp.ones((N_PROXY, NUM_CLASSES), dtype=jnp.float32)
        u, v_t = jax.lax.fori_loop(0, SINKHORN_ITERS, sink_iter, (u0, v0))

        vrow = jax.lax.dot_general(v_t, OH, (((1,), (0,)), ((), ())),
                                   preferred_element_type=jnp.float32)
        Q = u * jnp.exp(C * (1.0 / EPSILON)) * vrow
        qs = _sub_sum(Q)
        s_c = jax.lax.dot_general(qs, OH, (((1,), (1,)), ((), ())),
                                  preferred_element_type=jnp.float32)
        s_c = jnp.where(s_c > 0, s_c, 1.0)
        s_row = jax.lax.dot_general(s_c, OH, (((1,), (0,)), ((), ())),
                                    preferred_element_type=jnp.float32)
        W = Q / s_row

        iota16 = jax.lax.broadcasted_iota(jnp.int32, (N_PROXY, BATCH), 0)
        km = jnp.zeros((N_PROXY, BATCH), dtype=jnp.float32)
        for _k in range(K):
            cur = jnp.where(km > 0, -1.0, W)
            m = jnp.max(cur, axis=0, keepdims=True)
            idx = jnp.min(jnp.where(cur == m, iota16, N_PROXY + 1), axis=0,
                          keepdims=True)
            km = jnp.where(iota16 == idx, 1.0, km)

        x = C * (1.0 / TEMP)
        logits = x - _sub_lse(x)
        pos = _sub_sum(W * km * logits)
        neg = _sub_lse(logits)

        ones_b = jnp.ones((1, BATCH), dtype=jnp.float32)
        counts = jax.lax.dot_general(ones_b, OH, (((1,), (1,)), ((), ())),
                                     preferred_element_type=jnp.float32)
        inv_cnt = 1.0 / jnp.where(counts > 0, counts, 1.0)
        inv_row = jax.lax.dot_general(inv_cnt, OH, (((1,), (0,)), ((), ())),
                                      preferred_element_type=jnp.float32)
        mle = -jnp.sum((pos - neg) * inv_row)

        P = proxy_ref[...]
        sim = jnp.clip(jax.lax.dot_general(P, P, (((1,), (1,)), ((), ())),
                                           preferred_element_type=jnp.float32)
                       * (1.0 / TEMP), -10.0, 10.0)
        npx = NUM_CLASSES * N_PROXY
        rown = jax.lax.broadcasted_iota(jnp.int32, (npx, N_PROXY), 0)
        coln = jax.lax.broadcasted_iota(jnp.int32, (npx, N_PROXY), 1)
        acc = jnp.zeros((npx, 1), dtype=jnp.float32)
        ldiag = jnp.zeros((npx, 1), dtype=jnp.float32)
        for c2 in range(NUM_CLASSES):
            xb = sim[:, c2 * N_PROXY:(c2 + 1) * N_PROXY]
            lb = xb - _row_lse(xb)
            acc = acc + _row_lse(lb)
            own = (rown // N_PROXY == c2) & (rown % N_PROXY == coln)
            ldiag = ldiag + _rowsum(jnp.where(own, lb, 0.0))
        pc = jnp.sum(acc - ldiag) * (1.0 / N_PROXY)

        total = (mle + pc) / NUM_CLASSES
        out_ref[...] = jnp.broadcast_to(total, (1, 1)).astype(jnp.float32)


@jax.jit
def kernel(features, targets, proxy):
    oh = (targets[None, :] ==
          jnp.arange(NUM_CLASSES, dtype=targets.dtype)[:, None]
          ).astype(jnp.float32)
    out = pl.pallas_call(
        _mpl_kernel,
        grid=(NUM_TILES,),
        in_specs=[
            pl.BlockSpec((ROW_TILE, FEAT_DIM), lambda i: (i, 0)),
            pl.BlockSpec((NUM_CLASSES, ROW_TILE), lambda i: (0, i)),
            pl.BlockSpec((NUM_CLASSES * N_PROXY, FEAT_DIM), lambda i: (0, 0)),
        ],
        out_specs=pl.BlockSpec((1, 1), lambda i: (0, 0)),
        out_shape=jax.ShapeDtypeStruct((1, 1), jnp.float32),
        scratch_shapes=[
            pltpu.VMEM((N_PROXY, BATCH), jnp.float32),
            pltpu.VMEM((NUM_CLASSES, BATCH), jnp.float32),
        ],
    )(features, oh, proxy)
    return out[0, 0]
